# Initial kernel scaffold; baseline (speedup 1.0000x reference)
import functools

import jax
import jax.numpy as jnp
from jax import lax
from jax.experimental import pallas as pl
from jax.experimental.pallas import tpu as pltpu
from jax.experimental.pallas import tpu_sc as plsc

N = 50000
E = 3200000
CH = 1024
NCHUNKS = E // CH
ROWS_PER_SUB = N // 16
BT = 2000
H2N = 2 * N


def _tc1_body(x_ref, w1_ref, asad_ref, hsplit_ref, t1_ref):
    h = jnp.dot(x_ref[...], w1_ref[...], preferred_element_type=jnp.float32)
    hsplit_ref[0] = h[:, :32]
    hsplit_ref[1] = h[:, 32:]
    t1_ref[...] = jnp.dot(h, asad_ref[...], preferred_element_type=jnp.float32)


def _tc_stage1(x, W1, asad):
    grid = N // BT
    return pl.pallas_call(
        _tc1_body,
        grid=(grid,),
        in_specs=[
            pl.BlockSpec((BT, 3), lambda i: (i, 0)),
            pl.BlockSpec((3, 64), lambda i: (0, 0)),
            pl.BlockSpec((64, 16), lambda i: (0, 0)),
        ],
        out_specs=[
            pl.BlockSpec((2, BT, 32), lambda i: (0, i, 0)),
            pl.BlockSpec((BT, 16), lambda i: (i, 0)),
        ],
        out_shape=[
            jax.ShapeDtypeStruct((2, N, 32), jnp.float32),
            jax.ShapeDtypeStruct((N, 16), jnp.float32),
        ],
    )(x, W1, asad)


def _sc1_body(edge_hbm, t1_hbm, hsp_hbm, z_hbm, out_hbm,
              srcv, dstv, srcadj, dst2d, arows, brows, hrows, accrows,
              accsp, sem):
    c = lax.axis_index("c")
    s = lax.axis_index("s")

    pltpu.sync_copy(z_hbm, accsp.at[pl.ds(s * ROWS_PER_SUB, ROWS_PER_SUB)])
    plsc.subcore_barrier()

    cN = c * N
    nj = 195 + jnp.where(s < 5, 1, 0)

    def chunk_body(j, _):
        chunk = s + 16 * j
        base = chunk * CH
        pltpu.sync_copy(edge_hbm.at[0, pl.ds(base, CH)], srcv)
        pltpu.sync_copy(edge_hbm.at[1, pl.ds(base, CH)], dstv)
        for jj in range(CH // 128):
            pltpu.sync_copy(edge_hbm.at[1, pl.ds(base + 128 * jj, 128)],
                            dst2d.at[jj])

        def adj_body(i, _):
            srcadj[pl.ds(16 * i, 16)] = srcv[pl.ds(16 * i, 16)] + cN
            return ()
        lax.fori_loop(0, CH // 16, adj_body, ())

        cp1 = pltpu.async_copy(t1_hbm.at[srcv], arows, sem)
        cp2 = pltpu.async_copy(t1_hbm.at[dstv], brows, sem)
        cp3 = pltpu.async_copy(hsp_hbm.at[srcadj], hrows, sem)
        cp1.wait()
        cp2.wait()
        cp3.wait()

        def grp_body(g, _):
            r16 = lax.broadcasted_iota(jnp.int32, (16,), 0) + 16 * g
            for hh in range(4):
                ca = jnp.full((16,), hh, jnp.int32) + c * 4
                cb = jnp.full((16,), 8, jnp.int32) + ca
                va = plsc.load_gather(arows, [r16, ca])
                vb = plsc.load_gather(brows, [r16, cb])
                al = va + vb
                al = jnp.where(al > 0, al, 0.2 * al)
                ea = jnp.exp(al)
                plsc.store_scatter(accrows,
                                   [r16, jnp.full((16,), 32 + hh, jnp.int32)],
                                   ea)
                for k in range(8):
                    col = jnp.full((16,), hh * 8 + k, jnp.int32)
                    vh = plsc.load_gather(hrows, [r16, col])
                    plsc.store_scatter(accrows, [r16, col], ea * vh)
            return ()
        lax.fori_loop(0, CH // 16, grp_body, ())

        for jj in range(CH // 128):
            pltpu.sync_copy(accrows.at[pl.ds(128 * jj, 128)],
                            accsp.at[dst2d.at[jj]], add=True)
        return ()

    lax.fori_loop(0, nj, chunk_body, ())
    plsc.subcore_barrier()
    pltpu.sync_copy(accsp.at[pl.ds(s * ROWS_PER_SUB, ROWS_PER_SUB)],
                    out_hbm.at[c, pl.ds(s * ROWS_PER_SUB, ROWS_PER_SUB)])


def _sc_stage1(edge_index, t1, hsplit, zeros36):
    mesh = plsc.VectorSubcoreMesh(core_axis_name="c", subcore_axis_name="s")
    k = functools.partial(
        pl.kernel, mesh=mesh,
        out_type=jax.ShapeDtypeStruct((2, N, 36), jnp.float32),
        scratch_types=[
            pltpu.VMEM((CH,), jnp.int32),
            pltpu.VMEM((CH,), jnp.int32),
            pltpu.VMEM((CH,), jnp.int32),
            pltpu.VMEM((CH // 128, 128), jnp.int32),
            pltpu.VMEM((CH, 16), jnp.float32),
            pltpu.VMEM((CH, 16), jnp.float32),
            pltpu.VMEM((CH, 32), jnp.float32),
            pltpu.VMEM((CH, 36), jnp.float32),
            pltpu.VMEM_SHARED((N, 36), jnp.float32),
            pltpu.SemaphoreType.DMA,
        ],
    )(_sc1_body)
    return k(edge_index, t1, hsplit, zeros36)


def _tc3_body(acc_ref, b1_ref, w2p_ref, t2_ref):
    parts = []
    for c in range(2):
        numer = acc_ref[c, :, 0:32]
        denom = acc_ref[c, :, 32:36]
        d = jnp.broadcast_to(denom.reshape(BT, 4, 1), (BT, 4, 8)).reshape(BT, 32)
        parts.append(numer / (d + 1e-16))
    h1 = jnp.concatenate(parts, axis=1) + b1_ref[...]
    h1 = jnp.where(h1 > 0, h1, jnp.exp(h1) - 1.0)
    t2part = jnp.dot(h1, w2p_ref[...], preferred_element_type=jnp.float32)
    t2_ref[...] = jnp.concatenate(
        [t2part, jnp.zeros((BT, 12), jnp.float32)], axis=1)


def _tc_stage3(acc1, b1, w2p):
    grid = N // BT
    return pl.pallas_call(
        _tc3_body,
        grid=(grid,),
        in_specs=[
            pl.BlockSpec((2, BT, 36), lambda i: (0, i, 0)),
            pl.BlockSpec((1, 64), lambda i: (0, 0)),
            pl.BlockSpec((64, 4), lambda i: (0, 0)),
        ],
        out_specs=pl.BlockSpec((BT, 16), lambda i: (i, 0)),
        out_shape=jax.ShapeDtypeStruct((N, 16), jnp.float32),
    )(acc1, b1, w2p)


def _sc2_body(edge_hbm, t2_hbm, z_hbm, out_hbm,
              srcv, dstv, dst2d, tsrows, tdrows, accrows, accsp, sem):
    c = lax.axis_index("c")
    s = lax.axis_index("s")

    pltpu.sync_copy(z_hbm, accsp.at[pl.ds(s * ROWS_PER_SUB, ROWS_PER_SUB)])
    pltpu.sync_copy(z_hbm.at[pl.ds(0, CH)], accrows)
    plsc.subcore_barrier()

    wid = s * 2 + c
    nj = 97 + jnp.where(wid < 21, 1, 0)

    def chunk_body(j, _):
        chunk = wid + 32 * j
        base = chunk * CH
        pltpu.sync_copy(edge_hbm.at[0, pl.ds(base, CH)], srcv)
        pltpu.sync_copy(edge_hbm.at[1, pl.ds(base, CH)], dstv)
        for jj in range(CH // 128):
            pltpu.sync_copy(edge_hbm.at[1, pl.ds(base + 128 * jj, 128)],
                            dst2d.at[jj])
        cp1 = pltpu.async_copy(t2_hbm.at[srcv], tsrows, sem)
        cp2 = pltpu.async_copy(t2_hbm.at[dstv], tdrows, sem)
        cp1.wait()
        cp2.wait()

        def grp_body(g, _):
            r16 = lax.broadcasted_iota(jnp.int32, (16,), 0) + 16 * g
            as2 = plsc.load_gather(tsrows, [r16, jnp.full((16,), 2, jnp.int32)])
            ad2 = plsc.load_gather(tdrows, [r16, jnp.full((16,), 3, jnp.int32)])
            al = as2 + ad2
            al = jnp.where(al > 0, al, 0.2 * al)
            ea = jnp.exp(al)
            g0 = plsc.load_gather(tsrows, [r16, jnp.full((16,), 0, jnp.int32)])
            g1 = plsc.load_gather(tsrows, [r16, jnp.full((16,), 1, jnp.int32)])
            plsc.store_scatter(accrows, [r16, jnp.full((16,), 0, jnp.int32)],
                               ea * g0)
            plsc.store_scatter(accrows, [r16, jnp.full((16,), 1, jnp.int32)],
                               ea * g1)
            plsc.store_scatter(accrows, [r16, jnp.full((16,), 2, jnp.int32)],
                               ea)
            return ()
        lax.fori_loop(0, CH // 16, grp_body, ())

        for jj in range(CH // 128):
            pltpu.sync_copy(accrows.at[pl.ds(128 * jj, 128)],
                            accsp.at[dst2d.at[jj]], add=True)
        return ()

    lax.fori_loop(0, nj, chunk_body, ())
    plsc.subcore_barrier()
    pltpu.sync_copy(accsp.at[pl.ds(s * ROWS_PER_SUB, ROWS_PER_SUB)],
                    out_hbm.at[c, pl.ds(s * ROWS_PER_SUB, ROWS_PER_SUB)])


def _sc_stage2(edge_index, t2, zeros8):
    mesh = plsc.VectorSubcoreMesh(core_axis_name="c", subcore_axis_name="s")
    k = functools.partial(
        pl.kernel, mesh=mesh,
        out_type=jax.ShapeDtypeStruct((2, N, 8), jnp.float32),
        scratch_types=[
            pltpu.VMEM((CH,), jnp.int32),
            pltpu.VMEM((CH,), jnp.int32),
            pltpu.VMEM((CH // 128, 128), jnp.int32),
            pltpu.VMEM((CH, 16), jnp.float32),
            pltpu.VMEM((CH, 16), jnp.float32),
            pltpu.VMEM((CH, 8), jnp.float32),
            pltpu.VMEM_SHARED((N, 8), jnp.float32),
            pltpu.SemaphoreType.DMA,
        ],
    )(_sc2_body)
    return k(edge_index, t2, zeros8)


def _tc5_body(acc_ref, b2_ref, out_ref):
    numer = acc_ref[0, :, 0:2] + acc_ref[1, :, 0:2]
    denom = acc_ref[0, :, 2:3] + acc_ref[1, :, 2:3]
    o = numer / (denom + 1e-16) + b2_ref[...]
    m = jnp.max(o, axis=1, keepdims=True)
    z = o - m
    lse = jnp.log(jnp.sum(jnp.exp(z), axis=1, keepdims=True))
    out_ref[...] = z - lse


def _tc_stage5(acc2, b2):
    grid = N // BT
    return pl.pallas_call(
        _tc5_body,
        grid=(grid,),
        in_specs=[
            pl.BlockSpec((2, BT, 8), lambda i: (0, i, 0)),
            pl.BlockSpec((1, 2), lambda i: (0, 0)),
        ],
        out_specs=pl.BlockSpec((BT, 2), lambda i: (i, 0)),
        out_shape=jax.ShapeDtypeStruct((N, 2), jnp.float32),
    )(acc2, b2)


def kernel(x, edge_index, W1, att_src1, att_dst1, b1, W2, att_src2, att_dst2, b2):
    a_s = att_src1[0]
    a_d = att_dst1[0]
    eye = jnp.eye(8, dtype=jnp.float32)
    As = (eye[:, None, :] * a_s[:, :, None]).reshape(64, 8)
    Ad = (eye[:, None, :] * a_d[:, :, None]).reshape(64, 8)
    asad = jnp.concatenate([As, Ad], axis=1)

    as2 = att_src2[0, 0]
    ad2 = att_dst2[0, 0]
    P = jnp.stack([jnp.array([1.0, 0.0], jnp.float32),
                   jnp.array([0.0, 1.0], jnp.float32),
                   as2, ad2], axis=1)
    w2p = W2 @ P

    edge_index = edge_index.astype(jnp.int32)
    zeros36 = jnp.zeros((ROWS_PER_SUB, 36), jnp.float32)
    zeros8 = jnp.zeros((ROWS_PER_SUB, 8), jnp.float32)

    hsplit, t1 = _tc_stage1(x, W1, asad)
    hsplit = hsplit.reshape(H2N, 32)
    acc1 = _sc_stage1(edge_index, t1, hsplit, zeros36)
    t2 = _tc_stage3(acc1, b1.reshape(1, 64), w2p)
    acc2 = _sc_stage2(edge_index, t2, zeros8)
    return _tc_stage5(acc2, b2.reshape(1, 2))


# speedup vs baseline: 24.1528x; 24.1528x over previous
import functools

import jax
import jax.numpy as jnp
from jax import lax
from jax.experimental import pallas as pl
from jax.experimental.pallas import tpu as pltpu
from jax.experimental.pallas import tpu_sc as plsc

N = 50000
E = 3200000
CH1 = 32
CH2 = 1024
NPAD = 50048
ROWS_PER_SUB = NPAD // 16
BT = 2000
H2N = 2 * N


def _tc1_body(x_ref, w1_ref, asad_ref, q0_ref, q1_ref, q2_ref, q3_ref, t1_ref):
    h = jnp.dot(x_ref[...], w1_ref[...], preferred_element_type=jnp.float32)
    q0_ref[...] = h[:, 0:16]
    q1_ref[...] = h[:, 16:32]
    q2_ref[...] = h[:, 32:48]
    q3_ref[...] = h[:, 48:64]
    t1_ref[...] = jnp.dot(h, asad_ref[...], preferred_element_type=jnp.float32)


def _tc_stage1(x, W1, asad):
    grid = N // BT
    return pl.pallas_call(
        _tc1_body,
        grid=(grid,),
        in_specs=[
            pl.BlockSpec((BT, 3), lambda i: (i, 0)),
            pl.BlockSpec((3, 64), lambda i: (0, 0)),
            pl.BlockSpec((64, 16), lambda i: (0, 0)),
        ],
        out_specs=[pl.BlockSpec((BT, 16), lambda i: (i, 0))] * 5,
        out_shape=[jax.ShapeDtypeStruct((N, 16), jnp.float32)] * 5,
    )(x, W1, asad)


def _sc1_body(edge_hbm, t1_hbm, q0_hbm, q1_hbm, q2_hbm, q3_hbm, z8_hbm,
              on0_hbm, on1_hbm, on2_hbm, on3_hbm, od_hbm,
              srcv, dstv, dst2d, arows, brows, hrowsA, hrowsB,
              an0, an1, an2, an3, ad,
              spn0, spn1, spn2, spn3, spd, sem):
    c = lax.axis_index("c")
    s = lax.axis_index("s")

    for sp in (spn0, spn1, spn2, spn3, spd):
        pltpu.sync_copy(z8_hbm, sp.at[pl.ds(s * ROWS_PER_SUB, ROWS_PER_SUB)])
    plsc.subcore_barrier()

    nj = 6250
    accn = (an0, an1, an2, an3)

    def make_grp_body(hoff, hA, hB):
        def grp_body(g, _):
            r16 = lax.broadcasted_iota(jnp.int32, (16,), 0) + 16 * g
            for t in range(4):
                ca = jnp.full((16,), hoff + t, jnp.int32)
                cb = jnp.full((16,), 8 + hoff + t, jnp.int32)
                va = plsc.load_gather(arows, [r16, ca])
                vb = plsc.load_gather(brows, [r16, cb])
                al = va + vb
                al = jnp.where(al > 0, al, 0.2 * al)
                ea = jnp.exp(al)
                plsc.store_scatter(ad, [r16, jnp.full((16,), t, jnp.int32)], ea)
                hq = hA if t < 2 else hB
                for k in range(8):
                    cq = jnp.full((16,), (t % 2) * 8 + k, jnp.int32)
                    vh = plsc.load_gather(hq, [r16, cq])
                    plsc.store_scatter(accn[t],
                                       [r16, jnp.full((16,), k, jnp.int32)],
                                       ea * vh)
            return ()
        return grp_body

    def chunk_body(j, _):
        chunk = s + 16 * j
        base = chunk * CH1
        pltpu.sync_copy(edge_hbm.at[0, pl.ds(base, CH1)], srcv)
        pltpu.sync_copy(edge_hbm.at[1, pl.ds(base, CH1)], dstv)
        pltpu.sync_copy(edge_hbm.at[1, pl.ds(base, CH1)], dst2d.at[0])

        cp1 = pltpu.async_copy(t1_hbm.at[srcv], arows, sem)
        cp2 = pltpu.async_copy(t1_hbm.at[dstv], brows, sem)
        cp1.wait()
        cp2.wait()

        @pl.when(c == 0)
        def _():
            pltpu.async_copy(q0_hbm.at[srcv], hrowsA, sem).wait()
            pltpu.async_copy(q1_hbm.at[srcv], hrowsB, sem).wait()
            lax.fori_loop(0, CH1 // 16, make_grp_body(0, hrowsA, hrowsB), ())

        @pl.when(c == 1)
        def _():
            pltpu.async_copy(q2_hbm.at[srcv], hrowsA, sem).wait()
            pltpu.async_copy(q3_hbm.at[srcv], hrowsB, sem).wait()
            lax.fori_loop(0, CH1 // 16, make_grp_body(4, hrowsA, hrowsB), ())

        pltpu.sync_copy(an0, spn0.at[dst2d.at[0]], add=True)
        pltpu.sync_copy(an1, spn1.at[dst2d.at[0]], add=True)
        pltpu.sync_copy(an2, spn2.at[dst2d.at[0]], add=True)
        pltpu.sync_copy(an3, spn3.at[dst2d.at[0]], add=True)
        pltpu.sync_copy(ad, spd.at[dst2d.at[0]], add=True)
        return ()

    lax.fori_loop(0, nj, chunk_body, ())
    plsc.subcore_barrier()

    outs = (on0_hbm, on1_hbm, on2_hbm, on3_hbm, od_hbm)
    sps = (spn0, spn1, spn2, spn3, spd)

    @pl.when(c == 0)
    def _():
        for o, sp in zip(outs, sps):
            pltpu.sync_copy(sp.at[pl.ds(s * ROWS_PER_SUB, ROWS_PER_SUB)],
                            o.at[0, pl.ds(s * ROWS_PER_SUB, ROWS_PER_SUB)])

    @pl.when(c == 1)
    def _():
        for o, sp in zip(outs, sps):
            pltpu.sync_copy(sp.at[pl.ds(s * ROWS_PER_SUB, ROWS_PER_SUB)],
                            o.at[1, pl.ds(s * ROWS_PER_SUB, ROWS_PER_SUB)])


def _sc_stage1(edge_index, t1, q0, q1, q2, q3, zeros8):
    mesh = plsc.VectorSubcoreMesh(core_axis_name="c", subcore_axis_name="s")
    k = functools.partial(
        pl.kernel, mesh=mesh,
        compiler_params=pltpu.CompilerParams(needs_layout_passes=False, use_tc_tiling_on_sc=False),
        out_type=[jax.ShapeDtypeStruct((2, NPAD, 8), jnp.float32)] * 5,
        scratch_types=[
            pltpu.VMEM((CH1,), jnp.int32),
            pltpu.VMEM((CH1,), jnp.int32),
            pltpu.VMEM((1, CH1), jnp.int32),
            pltpu.VMEM((CH1, 16), jnp.float32),
            pltpu.VMEM((CH1, 16), jnp.float32),
            pltpu.VMEM((CH1, 16), jnp.float32),
            pltpu.VMEM((CH1, 16), jnp.float32),
            pltpu.VMEM((CH1, 8), jnp.float32),
            pltpu.VMEM((CH1, 8), jnp.float32),
            pltpu.VMEM((CH1, 8), jnp.float32),
            pltpu.VMEM((CH1, 8), jnp.float32),
            pltpu.VMEM((CH1, 8), jnp.float32),
            pltpu.VMEM_SHARED((NPAD, 8), jnp.float32),
            pltpu.VMEM_SHARED((NPAD, 8), jnp.float32),
            pltpu.VMEM_SHARED((NPAD, 8), jnp.float32),
            pltpu.VMEM_SHARED((NPAD, 8), jnp.float32),
            pltpu.VMEM_SHARED((NPAD, 8), jnp.float32),
            pltpu.SemaphoreType.DMA,
        ],
    )(_sc1_body)
    return k(edge_index, t1, q0, q1, q2, q3, zeros8)


def _tc3_body(n0_ref, n1_ref, n2_ref, n3_ref, d_ref, b1_ref, w2p_ref, t2_ref):
    nrefs = (n0_ref, n1_ref, n2_ref, n3_ref)
    parts = []
    for c in range(2):
        numer = jnp.concatenate([nr[c, :, :] for nr in nrefs], axis=1)
        denom = d_ref[c, :, 0:4]
        d = jnp.broadcast_to(denom.reshape(BT, 4, 1), (BT, 4, 8)).reshape(BT, 32)
        parts.append(numer / (d + 1e-16))
    h1 = jnp.concatenate(parts, axis=1) + b1_ref[...]
    h1 = jnp.where(h1 > 0, h1, jnp.exp(h1) - 1.0)
    t2part = jnp.dot(h1, w2p_ref[...], preferred_element_type=jnp.float32)
    t2_ref[...] = jnp.concatenate(
        [t2part, jnp.zeros((BT, 12), jnp.float32)], axis=1)


def _tc_stage3(accs, b1, w2p):
    grid = N // BT
    return pl.pallas_call(
        _tc3_body,
        grid=(grid,),
        in_specs=[pl.BlockSpec((2, BT, 8), lambda i: (0, i, 0))] * 5 + [
            pl.BlockSpec((1, 64), lambda i: (0, 0)),
            pl.BlockSpec((64, 4), lambda i: (0, 0)),
        ],
        out_specs=pl.BlockSpec((BT, 16), lambda i: (i, 0)),
        out_shape=jax.ShapeDtypeStruct((N, 16), jnp.float32),
    )(*accs, b1, w2p)


def _sc2_body(edge_hbm, t2_hbm, z_hbm, out_hbm,
              srcv, dstv, dst2d, tsrows, tdrows, accrows, accsp, sem):
    c = lax.axis_index("c")
    s = lax.axis_index("s")

    pltpu.sync_copy(z_hbm, accsp.at[pl.ds(s * ROWS_PER_SUB, ROWS_PER_SUB)])
    pltpu.sync_copy(z_hbm.at[pl.ds(0, CH2)], accrows)
    plsc.subcore_barrier()

    wid = s * 2 + c
    nj = 97 + jnp.where(wid < 21, 1, 0)

    def chunk_body(j, _):
        chunk = wid + 32 * j
        base = chunk * CH2
        pltpu.sync_copy(edge_hbm.at[0, pl.ds(base, CH2)], srcv)
        pltpu.sync_copy(edge_hbm.at[1, pl.ds(base, CH2)], dstv)
        for jj in range(CH2 // 128):
            pltpu.sync_copy(edge_hbm.at[1, pl.ds(base + 128 * jj, 128)],
                            dst2d.at[jj])
        cp1 = pltpu.async_copy(t2_hbm.at[srcv], tsrows, sem)
        cp2 = pltpu.async_copy(t2_hbm.at[dstv], tdrows, sem)
        cp1.wait()
        cp2.wait()

        def grp_body(g, _):
            r16 = lax.broadcasted_iota(jnp.int32, (16,), 0) + 16 * g
            as2 = plsc.load_gather(tsrows, [r16, jnp.full((16,), 2, jnp.int32)])
            ad2 = plsc.load_gather(tdrows, [r16, jnp.full((16,), 3, jnp.int32)])
            al = as2 + ad2
            al = jnp.where(al > 0, al, 0.2 * al)
            ea = jnp.exp(al)
            g0 = plsc.load_gather(tsrows, [r16, jnp.full((16,), 0, jnp.int32)])
            g1 = plsc.load_gather(tsrows, [r16, jnp.full((16,), 1, jnp.int32)])
            plsc.store_scatter(accrows, [r16, jnp.full((16,), 0, jnp.int32)],
                               ea * g0)
            plsc.store_scatter(accrows, [r16, jnp.full((16,), 1, jnp.int32)],
                               ea * g1)
            plsc.store_scatter(accrows, [r16, jnp.full((16,), 2, jnp.int32)],
                               ea)
            return ()
        lax.fori_loop(0, CH2 // 16, grp_body, ())

        for jj in range(CH2 // 128):
            pltpu.sync_copy(accrows.at[pl.ds(128 * jj, 128)],
                            accsp.at[dst2d.at[jj]], add=True)
        return ()

    lax.fori_loop(0, nj, chunk_body, ())
    plsc.subcore_barrier()

    @pl.when(c == 0)
    def _():
        pltpu.sync_copy(accsp.at[pl.ds(s * ROWS_PER_SUB, ROWS_PER_SUB)],
                        out_hbm.at[0, pl.ds(s * ROWS_PER_SUB, ROWS_PER_SUB)])

    @pl.when(c == 1)
    def _():
        pltpu.sync_copy(accsp.at[pl.ds(s * ROWS_PER_SUB, ROWS_PER_SUB)],
                        out_hbm.at[1, pl.ds(s * ROWS_PER_SUB, ROWS_PER_SUB)])


def _sc_stage2(edge_index, t2, zeros8):
    mesh = plsc.VectorSubcoreMesh(core_axis_name="c", subcore_axis_name="s")
    k = functools.partial(
        pl.kernel, mesh=mesh,
        compiler_params=pltpu.CompilerParams(needs_layout_passes=False, use_tc_tiling_on_sc=False),
        out_type=jax.ShapeDtypeStruct((2, NPAD, 8), jnp.float32),
        scratch_types=[
            pltpu.VMEM((CH2,), jnp.int32),
            pltpu.VMEM((CH2,), jnp.int32),
            pltpu.VMEM((CH2 // 128, 128), jnp.int32),
            pltpu.VMEM((CH2, 16), jnp.float32),
            pltpu.VMEM((CH2, 16), jnp.float32),
            pltpu.VMEM((CH2, 8), jnp.float32),
            pltpu.VMEM_SHARED((NPAD, 8), jnp.float32),
            pltpu.SemaphoreType.DMA,
        ],
    )(_sc2_body)
    return k(edge_index, t2, zeros8)


def _tc5_body(acc_ref, b2_ref, out_ref):
    numer = acc_ref[0, :, 0:2] + acc_ref[1, :, 0:2]
    denom = acc_ref[0, :, 2:3] + acc_ref[1, :, 2:3]
    o = numer / (denom + 1e-16) + b2_ref[...]
    m = jnp.max(o, axis=1, keepdims=True)
    z = o - m
    lse = jnp.log(jnp.sum(jnp.exp(z), axis=1, keepdims=True))
    out_ref[...] = z - lse


def _tc_stage5(acc2, b2):
    grid = N // BT
    return pl.pallas_call(
        _tc5_body,
        grid=(grid,),
        in_specs=[
            pl.BlockSpec((2, BT, 8), lambda i: (0, i, 0)),
            pl.BlockSpec((1, 2), lambda i: (0, 0)),
        ],
        out_specs=pl.BlockSpec((BT, 2), lambda i: (i, 0)),
        out_shape=jax.ShapeDtypeStruct((N, 2), jnp.float32),
    )(acc2, b2)


def kernel(x, edge_index, W1, att_src1, att_dst1, b1, W2, att_src2, att_dst2, b2):
    a_s = att_src1[0]
    a_d = att_dst1[0]
    eye = jnp.eye(8, dtype=jnp.float32)
    As = (eye[:, None, :] * a_s[:, :, None]).reshape(64, 8)
    Ad = (eye[:, None, :] * a_d[:, :, None]).reshape(64, 8)
    asad = jnp.concatenate([As, Ad], axis=1)

    as2 = att_src2[0, 0]
    ad2 = att_dst2[0, 0]
    P = jnp.stack([jnp.array([1.0, 0.0], jnp.float32),
                   jnp.array([0.0, 1.0], jnp.float32),
                   as2, ad2], axis=1)
    w2p = W2 @ P

    edge_index = edge_index.astype(jnp.int32)
    zeros8 = jnp.zeros((ROWS_PER_SUB, 8), jnp.float32)

    q0, q1, q2, q3, t1 = _tc_stage1(x, W1, asad)
    accs = _sc_stage1(edge_index, t1, q0, q1, q2, q3, zeros8)
    t2 = _tc_stage3(accs, b1.reshape(1, 64), w2p)
    acc2 = _sc_stage2(edge_index, t2, zeros8)
    return _tc_stage5(acc2, b2.reshape(1, 2))


# speedup vs baseline: 56.7892x; 2.3513x over previous
import functools

import jax
import jax.numpy as jnp
from jax import lax
from jax.experimental import pallas as pl
from jax.experimental.pallas import tpu as pltpu
from jax.experimental.pallas import tpu_sc as plsc

N = 50000
E = 3200000
CH1 = 512
CH2 = 1024
NPAD = 50048
ROWS_PER_SUB = NPAD // 16
BT = 2000
H2N = 2 * N


def _tc1_body(x_ref, w1_ref, asad_ref, q0_ref, q1_ref, q2_ref, q3_ref, t1_ref):
    h = jnp.dot(x_ref[...], w1_ref[...], preferred_element_type=jnp.float32)
    q0_ref[...] = h[:, 0:16]
    q1_ref[...] = h[:, 16:32]
    q2_ref[...] = h[:, 32:48]
    q3_ref[...] = h[:, 48:64]
    t1_ref[...] = jnp.dot(h, asad_ref[...], preferred_element_type=jnp.float32)


def _tc_stage1(x, W1, asad):
    grid = N // BT
    return pl.pallas_call(
        _tc1_body,
        grid=(grid,),
        in_specs=[
            pl.BlockSpec((BT, 3), lambda i: (i, 0)),
            pl.BlockSpec((3, 64), lambda i: (0, 0)),
            pl.BlockSpec((64, 16), lambda i: (0, 0)),
        ],
        out_specs=[pl.BlockSpec((BT, 16), lambda i: (i, 0))] * 5,
        out_shape=[jax.ShapeDtypeStruct((N, 16), jnp.float32)] * 5,
    )(x, W1, asad)


def _make_sc1_body(hbase):
    def body(edge_hbm, t1_hbm, qa_hbm, qb_hbm, z8_hbm,
             on0_hbm, on1_hbm, od_hbm,
             srcv, dstv, dst2d, arows, brows, hrows, an0, an1, ad,
             spn0, spn1, spd, sem):
        c = lax.axis_index("c")
        s = lax.axis_index("s")

        for sp in (spn0, spn1, spd):
            pltpu.sync_copy(z8_hbm, sp.at[pl.ds(s * ROWS_PER_SUB, ROWS_PER_SUB)])
        plsc.subcore_barrier()

        nj = 390 + jnp.where(s < 10, 1, 0)
        accn = (an0, an1)

        def make_grp_body(h0):
            def grp_body(g, _):
                r16 = lax.broadcasted_iota(jnp.int32, (16,), 0) + 16 * g
                for t in range(2):
                    ca = jnp.full((16,), h0 + t, jnp.int32)
                    cb = jnp.full((16,), 8 + h0 + t, jnp.int32)
                    va = plsc.load_gather(arows, [r16, ca])
                    vb = plsc.load_gather(brows, [r16, cb])
                    al = va + vb
                    al = jnp.where(al > 0, al, 0.2 * al)
                    ea = jnp.exp(al)
                    plsc.store_scatter(ad, [r16, jnp.full((16,), t, jnp.int32)], ea)
                    for k in range(8):
                        cq = jnp.full((16,), t * 8 + k, jnp.int32)
                        vh = plsc.load_gather(hrows, [r16, cq])
                        plsc.store_scatter(accn[t],
                                           [r16, jnp.full((16,), k, jnp.int32)],
                                           ea * vh)
                return ()
            return grp_body

        def chunk_body(j, _):
            chunk = s + 16 * j
            base = chunk * CH1
            pltpu.sync_copy(edge_hbm.at[0, pl.ds(base, CH1)], srcv)
            pltpu.sync_copy(edge_hbm.at[1, pl.ds(base, CH1)], dstv)
            for jj in range(CH1 // 128):
                pltpu.sync_copy(edge_hbm.at[1, pl.ds(base + 128 * jj, 128)],
                                dst2d.at[jj])

            cp1 = pltpu.async_copy(t1_hbm.at[srcv], arows, sem)
            cp2 = pltpu.async_copy(t1_hbm.at[dstv], brows, sem)
            cp1.wait()
            cp2.wait()

            @pl.when(c == 0)
            def _():
                pltpu.async_copy(qa_hbm.at[srcv], hrows, sem).wait()
                lax.fori_loop(0, CH1 // 16, make_grp_body(hbase), ())

            @pl.when(c == 1)
            def _():
                pltpu.async_copy(qb_hbm.at[srcv], hrows, sem).wait()
                lax.fori_loop(0, CH1 // 16, make_grp_body(hbase + 2), ())

            for jj in range(CH1 // 128):
                sl = pl.ds(128 * jj, 128)
                pltpu.sync_copy(an0.at[sl], spn0.at[dst2d.at[jj]], add=True)
                pltpu.sync_copy(an1.at[sl], spn1.at[dst2d.at[jj]], add=True)
                pltpu.sync_copy(ad.at[sl], spd.at[dst2d.at[jj]], add=True)
            return ()

        lax.fori_loop(0, nj, chunk_body, ())
        plsc.subcore_barrier()

        outs = (on0_hbm, on1_hbm, od_hbm)
        sps = (spn0, spn1, spd)

        @pl.when(c == 0)
        def _():
            for o, sp in zip(outs, sps):
                pltpu.sync_copy(sp.at[pl.ds(s * ROWS_PER_SUB, ROWS_PER_SUB)],
                                o.at[0, pl.ds(s * ROWS_PER_SUB, ROWS_PER_SUB)])

        @pl.when(c == 1)
        def _():
            for o, sp in zip(outs, sps):
                pltpu.sync_copy(sp.at[pl.ds(s * ROWS_PER_SUB, ROWS_PER_SUB)],
                                o.at[1, pl.ds(s * ROWS_PER_SUB, ROWS_PER_SUB)])

    return body


def _sc_stage1(edge_index, t1, qa, qb, zeros8, hbase):
    mesh = plsc.VectorSubcoreMesh(core_axis_name="c", subcore_axis_name="s")
    k = functools.partial(
        pl.kernel, mesh=mesh,
        compiler_params=pltpu.CompilerParams(needs_layout_passes=False, use_tc_tiling_on_sc=False),
        out_type=[jax.ShapeDtypeStruct((2, NPAD, 8), jnp.float32)] * 3,
        scratch_types=[
            pltpu.VMEM((CH1,), jnp.int32),
            pltpu.VMEM((CH1,), jnp.int32),
            pltpu.VMEM((CH1 // 128, 128), jnp.int32),
            pltpu.VMEM((CH1, 16), jnp.float32),
            pltpu.VMEM((CH1, 16), jnp.float32),
            pltpu.VMEM((CH1, 16), jnp.float32),
            pltpu.VMEM((CH1, 8), jnp.float32),
            pltpu.VMEM((CH1, 8), jnp.float32),
            pltpu.VMEM((CH1, 8), jnp.float32),
            pltpu.VMEM_SHARED((NPAD, 8), jnp.float32),
            pltpu.VMEM_SHARED((NPAD, 8), jnp.float32),
            pltpu.VMEM_SHARED((NPAD, 8), jnp.float32),
            pltpu.SemaphoreType.DMA,
        ],
    )(_make_sc1_body(hbase))
    return k(edge_index, t1, qa, qb, zeros8)


def _tc3_body(a0_ref, a1_ref, ad_ref, b0_ref, b1x_ref, bd_ref,
              b1_ref, w2p_ref, t2_ref):
    numer = jnp.concatenate([
        a0_ref[0], a1_ref[0], a0_ref[1], a1_ref[1],
        b0_ref[0], b1x_ref[0], b0_ref[1], b1x_ref[1]], axis=1)
    denom = jnp.concatenate([
        ad_ref[0, :, 0:1], ad_ref[0, :, 1:2], ad_ref[1, :, 0:1], ad_ref[1, :, 1:2],
        bd_ref[0, :, 0:1], bd_ref[0, :, 1:2], bd_ref[1, :, 0:1], bd_ref[1, :, 1:2]],
        axis=1)
    d = jnp.broadcast_to(denom.reshape(BT, 8, 1), (BT, 8, 8)).reshape(BT, 64)
    h1 = numer / (d + 1e-16) + b1_ref[...]
    h1 = jnp.where(h1 > 0, h1, jnp.exp(h1) - 1.0)
    t2part = jnp.dot(h1, w2p_ref[...], preferred_element_type=jnp.float32)
    t2_ref[...] = jnp.concatenate(
        [t2part, jnp.zeros((BT, 12), jnp.float32)], axis=1)


def _tc_stage3(accs, b1, w2p):
    grid = N // BT
    return pl.pallas_call(
        _tc3_body,
        grid=(grid,),
        in_specs=[pl.BlockSpec((2, BT, 8), lambda i: (0, i, 0))] * 6 + [
            pl.BlockSpec((1, 64), lambda i: (0, 0)),
            pl.BlockSpec((64, 4), lambda i: (0, 0)),
        ],
        out_specs=pl.BlockSpec((BT, 16), lambda i: (i, 0)),
        out_shape=jax.ShapeDtypeStruct((N, 16), jnp.float32),
    )(*accs, b1, w2p)


def _sc2_body(edge_hbm, t2_hbm, z_hbm, out_hbm,
              srcv, dstv, dst2d, tsrows, tdrows, accrows, accsp, sem):
    c = lax.axis_index("c")
    s = lax.axis_index("s")

    pltpu.sync_copy(z_hbm, accsp.at[pl.ds(s * ROWS_PER_SUB, ROWS_PER_SUB)])
    pltpu.sync_copy(z_hbm.at[pl.ds(0, CH2)], accrows)
    plsc.subcore_barrier()

    wid = s * 2 + c
    nj = 97 + jnp.where(wid < 21, 1, 0)

    def chunk_body(j, _):
        chunk = wid + 32 * j
        base = chunk * CH2
        pltpu.sync_copy(edge_hbm.at[0, pl.ds(base, CH2)], srcv)
        pltpu.sync_copy(edge_hbm.at[1, pl.ds(base, CH2)], dstv)
        for jj in range(CH2 // 128):
            pltpu.sync_copy(edge_hbm.at[1, pl.ds(base + 128 * jj, 128)],
                            dst2d.at[jj])
        cp1 = pltpu.async_copy(t2_hbm.at[srcv], tsrows, sem)
        cp2 = pltpu.async_copy(t2_hbm.at[dstv], tdrows, sem)
        cp1.wait()
        cp2.wait()

        def grp_body(g, _):
            r16 = lax.broadcasted_iota(jnp.int32, (16,), 0) + 16 * g
            as2 = plsc.load_gather(tsrows, [r16, jnp.full((16,), 2, jnp.int32)])
            ad2 = plsc.load_gather(tdrows, [r16, jnp.full((16,), 3, jnp.int32)])
            al = as2 + ad2
            al = jnp.where(al > 0, al, 0.2 * al)
            ea = jnp.exp(al)
            g0 = plsc.load_gather(tsrows, [r16, jnp.full((16,), 0, jnp.int32)])
            g1 = plsc.load_gather(tsrows, [r16, jnp.full((16,), 1, jnp.int32)])
            plsc.store_scatter(accrows, [r16, jnp.full((16,), 0, jnp.int32)],
                               ea * g0)
            plsc.store_scatter(accrows, [r16, jnp.full((16,), 1, jnp.int32)],
                               ea * g1)
            plsc.store_scatter(accrows, [r16, jnp.full((16,), 2, jnp.int32)],
                               ea)
            return ()
        lax.fori_loop(0, CH2 // 16, grp_body, ())

        for jj in range(CH2 // 128):
            pltpu.sync_copy(accrows.at[pl.ds(128 * jj, 128)],
                            accsp.at[dst2d.at[jj]], add=True)
        return ()

    lax.fori_loop(0, nj, chunk_body, ())
    plsc.subcore_barrier()

    @pl.when(c == 0)
    def _():
        pltpu.sync_copy(accsp.at[pl.ds(s * ROWS_PER_SUB, ROWS_PER_SUB)],
                        out_hbm.at[0, pl.ds(s * ROWS_PER_SUB, ROWS_PER_SUB)])

    @pl.when(c == 1)
    def _():
        pltpu.sync_copy(accsp.at[pl.ds(s * ROWS_PER_SUB, ROWS_PER_SUB)],
                        out_hbm.at[1, pl.ds(s * ROWS_PER_SUB, ROWS_PER_SUB)])


def _sc_stage2(edge_index, t2, zeros8):
    mesh = plsc.VectorSubcoreMesh(core_axis_name="c", subcore_axis_name="s")
    k = functools.partial(
        pl.kernel, mesh=mesh,
        compiler_params=pltpu.CompilerParams(needs_layout_passes=False, use_tc_tiling_on_sc=False),
        out_type=jax.ShapeDtypeStruct((2, NPAD, 8), jnp.float32),
        scratch_types=[
            pltpu.VMEM((CH2,), jnp.int32),
            pltpu.VMEM((CH2,), jnp.int32),
            pltpu.VMEM((CH2 // 128, 128), jnp.int32),
            pltpu.VMEM((CH2, 16), jnp.float32),
            pltpu.VMEM((CH2, 16), jnp.float32),
            pltpu.VMEM((CH2, 8), jnp.float32),
            pltpu.VMEM_SHARED((NPAD, 8), jnp.float32),
            pltpu.SemaphoreType.DMA,
        ],
    )(_sc2_body)
    return k(edge_index, t2, zeros8)


def _tc5_body(acc_ref, b2_ref, out_ref):
    numer = acc_ref[0, :, 0:2] + acc_ref[1, :, 0:2]
    denom = acc_ref[0, :, 2:3] + acc_ref[1, :, 2:3]
    o = numer / (denom + 1e-16) + b2_ref[...]
    m = jnp.max(o, axis=1, keepdims=True)
    z = o - m
    lse = jnp.log(jnp.sum(jnp.exp(z), axis=1, keepdims=True))
    out_ref[...] = z - lse


def _tc_stage5(acc2, b2):
    grid = N // BT
    return pl.pallas_call(
        _tc5_body,
        grid=(grid,),
        in_specs=[
            pl.BlockSpec((2, BT, 8), lambda i: (0, i, 0)),
            pl.BlockSpec((1, 2), lambda i: (0, 0)),
        ],
        out_specs=pl.BlockSpec((BT, 2), lambda i: (i, 0)),
        out_shape=jax.ShapeDtypeStruct((N, 2), jnp.float32),
    )(acc2, b2)


def kernel(x, edge_index, W1, att_src1, att_dst1, b1, W2, att_src2, att_dst2, b2):
    a_s = att_src1[0]
    a_d = att_dst1[0]
    eye = jnp.eye(8, dtype=jnp.float32)
    As = (eye[:, None, :] * a_s[:, :, None]).reshape(64, 8)
    Ad = (eye[:, None, :] * a_d[:, :, None]).reshape(64, 8)
    asad = jnp.concatenate([As, Ad], axis=1)

    as2 = att_src2[0, 0]
    ad2 = att_dst2[0, 0]
    P = jnp.stack([jnp.array([1.0, 0.0], jnp.float32),
                   jnp.array([0.0, 1.0], jnp.float32),
                   as2, ad2], axis=1)
    w2p = W2 @ P

    edge_index = edge_index.astype(jnp.int32)
    zeros8 = jnp.zeros((ROWS_PER_SUB, 8), jnp.float32)

    q0, q1, q2, q3, t1 = _tc_stage1(x, W1, asad)
    acc_a = _sc_stage1(edge_index, t1, q0, q1, zeros8, 0)
    acc_b = _sc_stage1(edge_index, t1, q2, q3, zeros8, 4)
    t2 = _tc_stage3(list(acc_a) + list(acc_b), b1.reshape(1, 64), w2p)
    acc2 = _sc_stage2(edge_index, t2, zeros8)
    return _tc_stage5(acc2, b2.reshape(1, 2))


# speedup vs baseline: 57.7088x; 1.0162x over previous
import functools

import jax
import jax.numpy as jnp
from jax import lax
from jax.experimental import pallas as pl
from jax.experimental.pallas import tpu as pltpu
from jax.experimental.pallas import tpu_sc as plsc

N = 50000
E = 3200000
CH1 = 512
CH2 = 1024
NPAD = 50048
ROWS_PER_SUB = NPAD // 16
BT = 2000
H2N = 2 * N


def _tc1_body(x_ref, w1_ref, asad_ref, q0_ref, q1_ref, q2_ref, q3_ref, t1_ref):
    h = jnp.dot(x_ref[...], w1_ref[...], preferred_element_type=jnp.float32)
    q0_ref[...] = h[:, 0:16]
    q1_ref[...] = h[:, 16:32]
    q2_ref[...] = h[:, 32:48]
    q3_ref[...] = h[:, 48:64]
    t1_ref[...] = jnp.dot(h, asad_ref[...], preferred_element_type=jnp.float32)


def _tc_stage1(x, W1, asad):
    grid = N // BT
    return pl.pallas_call(
        _tc1_body,
        grid=(grid,),
        in_specs=[
            pl.BlockSpec((BT, 3), lambda i: (i, 0)),
            pl.BlockSpec((3, 64), lambda i: (0, 0)),
            pl.BlockSpec((64, 16), lambda i: (0, 0)),
        ],
        out_specs=[pl.BlockSpec((BT, 16), lambda i: (i, 0))] * 5,
        out_shape=[jax.ShapeDtypeStruct((N, 16), jnp.float32)] * 5,
    )(x, W1, asad)


def _make_sc1_body(hbase):
    def body(edge_hbm, t1_hbm, qa_hbm, qb_hbm, z8_hbm,
             on01_hbm, od_hbm,
             srcv, dstv, dst2d, arows, brows, hrows, an01, ad,
             spn01, spd, sem):
        c = lax.axis_index("c")
        s = lax.axis_index("s")

        pltpu.sync_copy(z8_hbm, spn01.at[pl.ds(s * ROWS_PER_SUB, ROWS_PER_SUB), pl.ds(0, 8)])
        pltpu.sync_copy(z8_hbm, spn01.at[pl.ds(s * ROWS_PER_SUB, ROWS_PER_SUB), pl.ds(8, 8)])
        pltpu.sync_copy(z8_hbm, spd.at[pl.ds(s * ROWS_PER_SUB, ROWS_PER_SUB)])
        plsc.subcore_barrier()

        nj = 390 + jnp.where(s < 10, 1, 0)

        def make_grp_body(h0):
            def grp_body(g, _):
                r16 = lax.broadcasted_iota(jnp.int32, (16,), 0) + 16 * g
                for t in range(2):
                    ca = jnp.full((16,), h0 + t, jnp.int32)
                    cb = jnp.full((16,), 8 + h0 + t, jnp.int32)
                    va = plsc.load_gather(arows, [r16, ca])
                    vb = plsc.load_gather(brows, [r16, cb])
                    al = va + vb
                    al = jnp.where(al > 0, al, 0.2 * al)
                    ea = jnp.exp(al)
                    plsc.store_scatter(ad, [r16, jnp.full((16,), t, jnp.int32)], ea)
                    for k in range(8):
                        cq = jnp.full((16,), t * 8 + k, jnp.int32)
                        vh = plsc.load_gather(hrows, [r16, cq])
                        plsc.store_scatter(an01, [r16, cq], ea * vh)
                return ()
            return grp_body

        def chunk_body(j, _):
            chunk = s + 16 * j
            base = chunk * CH1
            pltpu.sync_copy(edge_hbm.at[0, pl.ds(base, CH1)], srcv)
            pltpu.sync_copy(edge_hbm.at[1, pl.ds(base, CH1)], dstv)
            for jj in range(CH1 // 128):
                pltpu.sync_copy(edge_hbm.at[1, pl.ds(base + 128 * jj, 128)],
                                dst2d.at[jj])

            cp1 = pltpu.async_copy(t1_hbm.at[srcv], arows, sem)
            cp2 = pltpu.async_copy(t1_hbm.at[dstv], brows, sem)
            cp1.wait()
            cp2.wait()

            @pl.when(c == 0)
            def _():
                pltpu.async_copy(qa_hbm.at[srcv], hrows, sem).wait()
                lax.fori_loop(0, CH1 // 16, make_grp_body(hbase), ())

            @pl.when(c == 1)
            def _():
                pltpu.async_copy(qb_hbm.at[srcv], hrows, sem).wait()
                lax.fori_loop(0, CH1 // 16, make_grp_body(hbase + 2), ())

            for jj in range(CH1 // 128):
                sl = pl.ds(128 * jj, 128)
                pltpu.sync_copy(an01.at[sl], spn01.at[dst2d.at[jj]], add=True)
                pltpu.sync_copy(ad.at[sl], spd.at[dst2d.at[jj]], add=True)
            return ()

        lax.fori_loop(0, nj, chunk_body, ())
        plsc.subcore_barrier()

        outs = (on01_hbm, od_hbm)
        sps = (spn01, spd)

        @pl.when(c == 0)
        def _():
            for o, sp in zip(outs, sps):
                pltpu.sync_copy(sp.at[pl.ds(s * ROWS_PER_SUB, ROWS_PER_SUB)],
                                o.at[0, pl.ds(s * ROWS_PER_SUB, ROWS_PER_SUB)])

        @pl.when(c == 1)
        def _():
            for o, sp in zip(outs, sps):
                pltpu.sync_copy(sp.at[pl.ds(s * ROWS_PER_SUB, ROWS_PER_SUB)],
                                o.at[1, pl.ds(s * ROWS_PER_SUB, ROWS_PER_SUB)])

    return body


def _sc_stage1(edge_index, t1, qa, qb, zeros8, hbase):
    mesh = plsc.VectorSubcoreMesh(core_axis_name="c", subcore_axis_name="s")
    k = functools.partial(
        pl.kernel, mesh=mesh,
        compiler_params=pltpu.CompilerParams(needs_layout_passes=False, use_tc_tiling_on_sc=False),
        out_type=[jax.ShapeDtypeStruct((2, NPAD, 16), jnp.float32),
                  jax.ShapeDtypeStruct((2, NPAD, 8), jnp.float32)],
        scratch_types=[
            pltpu.VMEM((CH1,), jnp.int32),
            pltpu.VMEM((CH1,), jnp.int32),
            pltpu.VMEM((CH1 // 128, 128), jnp.int32),
            pltpu.VMEM((CH1, 16), jnp.float32),
            pltpu.VMEM((CH1, 16), jnp.float32),
            pltpu.VMEM((CH1, 16), jnp.float32),
            pltpu.VMEM((CH1, 16), jnp.float32),
            pltpu.VMEM((CH1, 8), jnp.float32),
            pltpu.VMEM_SHARED((NPAD, 16), jnp.float32),
            pltpu.VMEM_SHARED((NPAD, 8), jnp.float32),
            pltpu.SemaphoreType.DMA,
        ],
    )(_make_sc1_body(hbase))
    return k(edge_index, t1, qa, qb, zeros8)


def _tc3_body(a01_ref, ad_ref, b01_ref, bd_ref, b1_ref, w2p_ref, t2_ref):
    numer = jnp.concatenate([
        a01_ref[0], a01_ref[1], b01_ref[0], b01_ref[1]], axis=1)
    denom = jnp.concatenate([
        ad_ref[0, :, 0:1], ad_ref[0, :, 1:2], ad_ref[1, :, 0:1], ad_ref[1, :, 1:2],
        bd_ref[0, :, 0:1], bd_ref[0, :, 1:2], bd_ref[1, :, 0:1], bd_ref[1, :, 1:2]],
        axis=1)
    d = jnp.broadcast_to(denom.reshape(BT, 8, 1), (BT, 8, 8)).reshape(BT, 64)
    h1 = numer / (d + 1e-16) + b1_ref[...]
    h1 = jnp.where(h1 > 0, h1, jnp.exp(h1) - 1.0)
    t2part = jnp.dot(h1, w2p_ref[...], preferred_element_type=jnp.float32)
    t2_ref[...] = jnp.concatenate(
        [t2part, jnp.zeros((BT, 12), jnp.float32)], axis=1)


def _tc_stage3(accs, b1, w2p):
    grid = N // BT
    return pl.pallas_call(
        _tc3_body,
        grid=(grid,),
        in_specs=[pl.BlockSpec((2, BT, 16), lambda i: (0, i, 0)),
                  pl.BlockSpec((2, BT, 8), lambda i: (0, i, 0)),
                  pl.BlockSpec((2, BT, 16), lambda i: (0, i, 0)),
                  pl.BlockSpec((2, BT, 8), lambda i: (0, i, 0))] + [
            pl.BlockSpec((1, 64), lambda i: (0, 0)),
            pl.BlockSpec((64, 4), lambda i: (0, 0)),
        ],
        out_specs=pl.BlockSpec((BT, 16), lambda i: (i, 0)),
        out_shape=jax.ShapeDtypeStruct((N, 16), jnp.float32),
    )(*accs, b1, w2p)


def _sc2_body(edge_hbm, t2_hbm, z_hbm, out_hbm,
              srcv, dstv, dst2d, tsrows, tdrows, accrows, accsp, sem):
    c = lax.axis_index("c")
    s = lax.axis_index("s")

    pltpu.sync_copy(z_hbm, accsp.at[pl.ds(s * ROWS_PER_SUB, ROWS_PER_SUB)])
    pltpu.sync_copy(z_hbm.at[pl.ds(0, CH2)], accrows)
    plsc.subcore_barrier()

    wid = s * 2 + c
    nj = 97 + jnp.where(wid < 21, 1, 0)

    def chunk_body(j, _):
        chunk = wid + 32 * j
        base = chunk * CH2
        pltpu.sync_copy(edge_hbm.at[0, pl.ds(base, CH2)], srcv)
        pltpu.sync_copy(edge_hbm.at[1, pl.ds(base, CH2)], dstv)
        for jj in range(CH2 // 128):
            pltpu.sync_copy(edge_hbm.at[1, pl.ds(base + 128 * jj, 128)],
                            dst2d.at[jj])
        cp1 = pltpu.async_copy(t2_hbm.at[srcv], tsrows, sem)
        cp2 = pltpu.async_copy(t2_hbm.at[dstv], tdrows, sem)
        cp1.wait()
        cp2.wait()

        def grp_body(g, _):
            r16 = lax.broadcasted_iota(jnp.int32, (16,), 0) + 16 * g
            as2 = plsc.load_gather(tsrows, [r16, jnp.full((16,), 2, jnp.int32)])
            ad2 = plsc.load_gather(tdrows, [r16, jnp.full((16,), 3, jnp.int32)])
            al = as2 + ad2
            al = jnp.where(al > 0, al, 0.2 * al)
            ea = jnp.exp(al)
            g0 = plsc.load_gather(tsrows, [r16, jnp.full((16,), 0, jnp.int32)])
            g1 = plsc.load_gather(tsrows, [r16, jnp.full((16,), 1, jnp.int32)])
            plsc.store_scatter(accrows, [r16, jnp.full((16,), 0, jnp.int32)],
                               ea * g0)
            plsc.store_scatter(accrows, [r16, jnp.full((16,), 1, jnp.int32)],
                               ea * g1)
            plsc.store_scatter(accrows, [r16, jnp.full((16,), 2, jnp.int32)],
                               ea)
            return ()
        lax.fori_loop(0, CH2 // 16, grp_body, ())

        for jj in range(CH2 // 128):
            pltpu.sync_copy(accrows.at[pl.ds(128 * jj, 128)],
                            accsp.at[dst2d.at[jj]], add=True)
        return ()

    lax.fori_loop(0, nj, chunk_body, ())
    plsc.subcore_barrier()

    @pl.when(c == 0)
    def _():
        pltpu.sync_copy(accsp.at[pl.ds(s * ROWS_PER_SUB, ROWS_PER_SUB)],
                        out_hbm.at[0, pl.ds(s * ROWS_PER_SUB, ROWS_PER_SUB)])

    @pl.when(c == 1)
    def _():
        pltpu.sync_copy(accsp.at[pl.ds(s * ROWS_PER_SUB, ROWS_PER_SUB)],
                        out_hbm.at[1, pl.ds(s * ROWS_PER_SUB, ROWS_PER_SUB)])


def _sc_stage2(edge_index, t2, zeros8):
    mesh = plsc.VectorSubcoreMesh(core_axis_name="c", subcore_axis_name="s")
    k = functools.partial(
        pl.kernel, mesh=mesh,
        compiler_params=pltpu.CompilerParams(needs_layout_passes=False, use_tc_tiling_on_sc=False),
        out_type=jax.ShapeDtypeStruct((2, NPAD, 8), jnp.float32),
        scratch_types=[
            pltpu.VMEM((CH2,), jnp.int32),
            pltpu.VMEM((CH2,), jnp.int32),
            pltpu.VMEM((CH2 // 128, 128), jnp.int32),
            pltpu.VMEM((CH2, 16), jnp.float32),
            pltpu.VMEM((CH2, 16), jnp.float32),
            pltpu.VMEM((CH2, 8), jnp.float32),
            pltpu.VMEM_SHARED((NPAD, 8), jnp.float32),
            pltpu.SemaphoreType.DMA,
        ],
    )(_sc2_body)
    return k(edge_index, t2, zeros8)


def _tc5_body(acc_ref, b2_ref, out_ref):
    numer = acc_ref[0, :, 0:2] + acc_ref[1, :, 0:2]
    denom = acc_ref[0, :, 2:3] + acc_ref[1, :, 2:3]
    o = numer / (denom + 1e-16) + b2_ref[...]
    m = jnp.max(o, axis=1, keepdims=True)
    z = o - m
    lse = jnp.log(jnp.sum(jnp.exp(z), axis=1, keepdims=True))
    out_ref[...] = z - lse


def _tc_stage5(acc2, b2):
    grid = N // BT
    return pl.pallas_call(
        _tc5_body,
        grid=(grid,),
        in_specs=[
            pl.BlockSpec((2, BT, 8), lambda i: (0, i, 0)),
            pl.BlockSpec((1, 2), lambda i: (0, 0)),
        ],
        out_specs=pl.BlockSpec((BT, 2), lambda i: (i, 0)),
        out_shape=jax.ShapeDtypeStruct((N, 2), jnp.float32),
    )(acc2, b2)


def kernel(x, edge_index, W1, att_src1, att_dst1, b1, W2, att_src2, att_dst2, b2):
    a_s = att_src1[0]
    a_d = att_dst1[0]
    eye = jnp.eye(8, dtype=jnp.float32)
    As = (eye[:, None, :] * a_s[:, :, None]).reshape(64, 8)
    Ad = (eye[:, None, :] * a_d[:, :, None]).reshape(64, 8)
    asad = jnp.concatenate([As, Ad], axis=1)

    as2 = att_src2[0, 0]
    ad2 = att_dst2[0, 0]
    P = jnp.stack([jnp.array([1.0, 0.0], jnp.float32),
                   jnp.array([0.0, 1.0], jnp.float32),
                   as2, ad2], axis=1)
    w2p = W2 @ P

    edge_index = edge_index.astype(jnp.int32)
    zeros8 = jnp.zeros((ROWS_PER_SUB, 8), jnp.float32)

    q0, q1, q2, q3, t1 = _tc_stage1(x, W1, asad)
    acc_a = _sc_stage1(edge_index, t1, q0, q1, zeros8, 0)
    acc_b = _sc_stage1(edge_index, t1, q2, q3, zeros8, 4)
    t2 = _tc_stage3(list(acc_a) + list(acc_b), b1.reshape(1, 64), w2p)
    acc2 = _sc_stage2(edge_index, t2, zeros8)
    return _tc_stage5(acc2, b2.reshape(1, 2))


# speedup vs baseline: 59.4009x; 1.0293x over previous
import functools

import jax
import jax.numpy as jnp
from jax import lax
from jax.experimental import pallas as pl
from jax.experimental.pallas import tpu as pltpu
from jax.experimental.pallas import tpu_sc as plsc

N = 50000
E = 3200000
CH1 = 640
CH2 = 1024
NPAD = 50048
ROWS_PER_SUB = NPAD // 16
BT = 2000
H2N = 2 * N


def _tc1_body(x_ref, w1_ref, asad_ref, q0_ref, q1_ref, q2_ref, q3_ref, t1_ref):
    h = jnp.dot(x_ref[...], w1_ref[...], preferred_element_type=jnp.float32)
    q0_ref[...] = h[:, 0:16]
    q1_ref[...] = h[:, 16:32]
    q2_ref[...] = h[:, 32:48]
    q3_ref[...] = h[:, 48:64]
    t1_ref[...] = jnp.dot(h, asad_ref[...], preferred_element_type=jnp.float32)


def _tc_stage1(x, W1, asad):
    grid = N // BT
    return pl.pallas_call(
        _tc1_body,
        grid=(grid,),
        in_specs=[
            pl.BlockSpec((BT, 3), lambda i: (i, 0)),
            pl.BlockSpec((3, 64), lambda i: (0, 0)),
            pl.BlockSpec((64, 16), lambda i: (0, 0)),
        ],
        out_specs=[pl.BlockSpec((BT, 16), lambda i: (i, 0))] * 5,
        out_shape=[jax.ShapeDtypeStruct((N, 16), jnp.float32)] * 5,
    )(x, W1, asad)


def _make_sc1_body(hbase):
    def body(edge_hbm, t1_hbm, qa_hbm, qb_hbm, z8_hbm,
             on01_hbm, od_hbm,
             srcv, dstv, dst2d, arows, brows, hrows, an01, ad,
             spn01, spd, sem):
        c = lax.axis_index("c")
        s = lax.axis_index("s")

        pltpu.sync_copy(z8_hbm, spn01.at[pl.ds(s * ROWS_PER_SUB, ROWS_PER_SUB), pl.ds(0, 8)])
        pltpu.sync_copy(z8_hbm, spn01.at[pl.ds(s * ROWS_PER_SUB, ROWS_PER_SUB), pl.ds(8, 8)])
        pltpu.sync_copy(z8_hbm, spd.at[pl.ds(s * ROWS_PER_SUB, ROWS_PER_SUB)])
        plsc.subcore_barrier()

        nj = 312 + jnp.where(s < 8, 1, 0)

        def make_grp_body(h0):
            def grp_body(g, _):
                r16 = lax.broadcasted_iota(jnp.int32, (16,), 0) + 16 * g
                for t in range(2):
                    ca = jnp.full((16,), h0 + t, jnp.int32)
                    cb = jnp.full((16,), 8 + h0 + t, jnp.int32)
                    va = plsc.load_gather(arows, [r16, ca])
                    vb = plsc.load_gather(brows, [r16, cb])
                    al = va + vb
                    al = jnp.where(al > 0, al, 0.2 * al)
                    ea = jnp.exp(al)
                    plsc.store_scatter(ad, [r16, jnp.full((16,), t, jnp.int32)], ea)
                    for k in range(8):
                        cq = jnp.full((16,), t * 8 + k, jnp.int32)
                        vh = plsc.load_gather(hrows, [r16, cq])
                        plsc.store_scatter(an01, [r16, cq], ea * vh)
                return ()
            return grp_body

        def chunk_body(j, _):
            chunk = s + 16 * j
            base = chunk * CH1
            pltpu.sync_copy(edge_hbm.at[0, pl.ds(base, CH1)], srcv)
            pltpu.sync_copy(edge_hbm.at[1, pl.ds(base, CH1)], dstv)
            for jj in range(CH1 // 128):
                pltpu.sync_copy(edge_hbm.at[1, pl.ds(base + 128 * jj, 128)],
                                dst2d.at[jj])

            cp1 = pltpu.async_copy(t1_hbm.at[srcv], arows, sem)
            cp2 = pltpu.async_copy(t1_hbm.at[dstv], brows, sem)
            cp1.wait()
            cp2.wait()

            @pl.when(c == 0)
            def _():
                pltpu.async_copy(qa_hbm.at[srcv], hrows, sem).wait()
                lax.fori_loop(0, CH1 // 16, make_grp_body(hbase), ())

            @pl.when(c == 1)
            def _():
                pltpu.async_copy(qb_hbm.at[srcv], hrows, sem).wait()
                lax.fori_loop(0, CH1 // 16, make_grp_body(hbase + 2), ())

            for jj in range(CH1 // 128):
                sl = pl.ds(128 * jj, 128)
                pltpu.sync_copy(an01.at[sl], spn01.at[dst2d.at[jj]], add=True)
                pltpu.sync_copy(ad.at[sl], spd.at[dst2d.at[jj]], add=True)
            return ()

        lax.fori_loop(0, nj, chunk_body, ())
        plsc.subcore_barrier()

        outs = (on01_hbm, od_hbm)
        sps = (spn01, spd)

        @pl.when(c == 0)
        def _():
            for o, sp in zip(outs, sps):
                pltpu.sync_copy(sp.at[pl.ds(s * ROWS_PER_SUB, ROWS_PER_SUB)],
                                o.at[0, pl.ds(s * ROWS_PER_SUB, ROWS_PER_SUB)])

        @pl.when(c == 1)
        def _():
            for o, sp in zip(outs, sps):
                pltpu.sync_copy(sp.at[pl.ds(s * ROWS_PER_SUB, ROWS_PER_SUB)],
                                o.at[1, pl.ds(s * ROWS_PER_SUB, ROWS_PER_SUB)])

    return body


def _sc_stage1(edge_index, t1, qa, qb, zeros8, hbase):
    mesh = plsc.VectorSubcoreMesh(core_axis_name="c", subcore_axis_name="s")
    k = functools.partial(
        pl.kernel, mesh=mesh,
        compiler_params=pltpu.CompilerParams(needs_layout_passes=False, use_tc_tiling_on_sc=False),
        out_type=[jax.ShapeDtypeStruct((2, NPAD, 16), jnp.float32),
                  jax.ShapeDtypeStruct((2, NPAD, 8), jnp.float32)],
        scratch_types=[
            pltpu.VMEM((CH1,), jnp.int32),
            pltpu.VMEM((CH1,), jnp.int32),
            pltpu.VMEM((CH1 // 128, 128), jnp.int32),
            pltpu.VMEM((CH1, 16), jnp.float32),
            pltpu.VMEM((CH1, 16), jnp.float32),
            pltpu.VMEM((CH1, 16), jnp.float32),
            pltpu.VMEM((CH1, 16), jnp.float32),
            pltpu.VMEM((CH1, 8), jnp.float32),
            pltpu.VMEM_SHARED((NPAD, 16), jnp.float32),
            pltpu.VMEM_SHARED((NPAD, 8), jnp.float32),
            pltpu.SemaphoreType.DMA,
        ],
    )(_make_sc1_body(hbase))
    return k(edge_index, t1, qa, qb, zeros8)


def _tc3_body(a01_ref, ad_ref, b01_ref, bd_ref, b1_ref, w2p_ref, t2_ref):
    numer = jnp.concatenate([
        a01_ref[0], a01_ref[1], b01_ref[0], b01_ref[1]], axis=1)
    denom = jnp.concatenate([
        ad_ref[0, :, 0:1], ad_ref[0, :, 1:2], ad_ref[1, :, 0:1], ad_ref[1, :, 1:2],
        bd_ref[0, :, 0:1], bd_ref[0, :, 1:2], bd_ref[1, :, 0:1], bd_ref[1, :, 1:2]],
        axis=1)
    d = jnp.broadcast_to(denom.reshape(BT, 8, 1), (BT, 8, 8)).reshape(BT, 64)
    h1 = numer / (d + 1e-16) + b1_ref[...]
    h1 = jnp.where(h1 > 0, h1, jnp.exp(h1) - 1.0)
    t2part = jnp.dot(h1, w2p_ref[...], preferred_element_type=jnp.float32)
    t2_ref[...] = jnp.concatenate(
        [t2part, jnp.zeros((BT, 12), jnp.float32)], axis=1)


def _tc_stage3(accs, b1, w2p):
    grid = N // BT
    return pl.pallas_call(
        _tc3_body,
        grid=(grid,),
        in_specs=[pl.BlockSpec((2, BT, 16), lambda i: (0, i, 0)),
                  pl.BlockSpec((2, BT, 8), lambda i: (0, i, 0)),
                  pl.BlockSpec((2, BT, 16), lambda i: (0, i, 0)),
                  pl.BlockSpec((2, BT, 8), lambda i: (0, i, 0))] + [
            pl.BlockSpec((1, 64), lambda i: (0, 0)),
            pl.BlockSpec((64, 4), lambda i: (0, 0)),
        ],
        out_specs=pl.BlockSpec((BT, 16), lambda i: (i, 0)),
        out_shape=jax.ShapeDtypeStruct((N, 16), jnp.float32),
    )(*accs, b1, w2p)


def _sc2_body(edge_hbm, t2_hbm, z_hbm, out_hbm,
              srcv, dstv, dst2d, tsrows, tdrows, accrows, accsp, sem):
    c = lax.axis_index("c")
    s = lax.axis_index("s")

    pltpu.sync_copy(z_hbm, accsp.at[pl.ds(s * ROWS_PER_SUB, ROWS_PER_SUB)])
    pltpu.sync_copy(z_hbm.at[pl.ds(0, CH2)], accrows)
    plsc.subcore_barrier()

    wid = s * 2 + c
    nj = 97 + jnp.where(wid < 21, 1, 0)

    def chunk_body(j, _):
        chunk = wid + 32 * j
        base = chunk * CH2
        pltpu.sync_copy(edge_hbm.at[0, pl.ds(base, CH2)], srcv)
        pltpu.sync_copy(edge_hbm.at[1, pl.ds(base, CH2)], dstv)
        for jj in range(CH2 // 128):
            pltpu.sync_copy(edge_hbm.at[1, pl.ds(base + 128 * jj, 128)],
                            dst2d.at[jj])
        cp1 = pltpu.async_copy(t2_hbm.at[srcv], tsrows, sem)
        cp2 = pltpu.async_copy(t2_hbm.at[dstv], tdrows, sem)
        cp1.wait()
        cp2.wait()

        def grp_body(g, _):
            r16 = lax.broadcasted_iota(jnp.int32, (16,), 0) + 16 * g
            as2 = plsc.load_gather(tsrows, [r16, jnp.full((16,), 2, jnp.int32)])
            ad2 = plsc.load_gather(tdrows, [r16, jnp.full((16,), 3, jnp.int32)])
            al = as2 + ad2
            al = jnp.where(al > 0, al, 0.2 * al)
            ea = jnp.exp(al)
            g0 = plsc.load_gather(tsrows, [r16, jnp.full((16,), 0, jnp.int32)])
            g1 = plsc.load_gather(tsrows, [r16, jnp.full((16,), 1, jnp.int32)])
            plsc.store_scatter(accrows, [r16, jnp.full((16,), 0, jnp.int32)],
                               ea * g0)
            plsc.store_scatter(accrows, [r16, jnp.full((16,), 1, jnp.int32)],
                               ea * g1)
            plsc.store_scatter(accrows, [r16, jnp.full((16,), 2, jnp.int32)],
                               ea)
            return ()
        lax.fori_loop(0, CH2 // 16, grp_body, ())

        for jj in range(CH2 // 128):
            pltpu.sync_copy(accrows.at[pl.ds(128 * jj, 128)],
                            accsp.at[dst2d.at[jj]], add=True)
        return ()

    lax.fori_loop(0, nj, chunk_body, ())
    plsc.subcore_barrier()

    @pl.when(c == 0)
    def _():
        pltpu.sync_copy(accsp.at[pl.ds(s * ROWS_PER_SUB, ROWS_PER_SUB)],
                        out_hbm.at[0, pl.ds(s * ROWS_PER_SUB, ROWS_PER_SUB)])

    @pl.when(c == 1)
    def _():
        pltpu.sync_copy(accsp.at[pl.ds(s * ROWS_PER_SUB, ROWS_PER_SUB)],
                        out_hbm.at[1, pl.ds(s * ROWS_PER_SUB, ROWS_PER_SUB)])


def _sc_stage2(edge_index, t2, zeros8):
    mesh = plsc.VectorSubcoreMesh(core_axis_name="c", subcore_axis_name="s")
    k = functools.partial(
        pl.kernel, mesh=mesh,
        compiler_params=pltpu.CompilerParams(needs_layout_passes=False, use_tc_tiling_on_sc=False),
        out_type=jax.ShapeDtypeStruct((2, NPAD, 8), jnp.float32),
        scratch_types=[
            pltpu.VMEM((CH2,), jnp.int32),
            pltpu.VMEM((CH2,), jnp.int32),
            pltpu.VMEM((CH2 // 128, 128), jnp.int32),
            pltpu.VMEM((CH2, 16), jnp.float32),
            pltpu.VMEM((CH2, 16), jnp.float32),
            pltpu.VMEM((CH2, 8), jnp.float32),
            pltpu.VMEM_SHARED((NPAD, 8), jnp.float32),
            pltpu.SemaphoreType.DMA,
        ],
    )(_sc2_body)
    return k(edge_index, t2, zeros8)


def _tc5_body(acc_ref, b2_ref, out_ref):
    numer = acc_ref[0, :, 0:2] + acc_ref[1, :, 0:2]
    denom = acc_ref[0, :, 2:3] + acc_ref[1, :, 2:3]
    o = numer / (denom + 1e-16) + b2_ref[...]
    m = jnp.max(o, axis=1, keepdims=True)
    z = o - m
    lse = jnp.log(jnp.sum(jnp.exp(z), axis=1, keepdims=True))
    out_ref[...] = z - lse


def _tc_stage5(acc2, b2):
    grid = N // BT
    return pl.pallas_call(
        _tc5_body,
        grid=(grid,),
        in_specs=[
            pl.BlockSpec((2, BT, 8), lambda i: (0, i, 0)),
            pl.BlockSpec((1, 2), lambda i: (0, 0)),
        ],
        out_specs=pl.BlockSpec((BT, 2), lambda i: (i, 0)),
        out_shape=jax.ShapeDtypeStruct((N, 2), jnp.float32),
    )(acc2, b2)


def kernel(x, edge_index, W1, att_src1, att_dst1, b1, W2, att_src2, att_dst2, b2):
    a_s = att_src1[0]
    a_d = att_dst1[0]
    eye = jnp.eye(8, dtype=jnp.float32)
    As = (eye[:, None, :] * a_s[:, :, None]).reshape(64, 8)
    Ad = (eye[:, None, :] * a_d[:, :, None]).reshape(64, 8)
    asad = jnp.concatenate([As, Ad], axis=1)

    as2 = att_src2[0, 0]
    ad2 = att_dst2[0, 0]
    P = jnp.stack([jnp.array([1.0, 0.0], jnp.float32),
                   jnp.array([0.0, 1.0], jnp.float32),
                   as2, ad2], axis=1)
    w2p = W2 @ P

    edge_index = edge_index.astype(jnp.int32)
    zeros8 = jnp.zeros((ROWS_PER_SUB, 8), jnp.float32)

    q0, q1, q2, q3, t1 = _tc_stage1(x, W1, asad)
    acc_a = _sc_stage1(edge_index, t1, q0, q1, zeros8, 0)
    acc_b = _sc_stage1(edge_index, t1, q2, q3, zeros8, 4)
    t2 = _tc_stage3(list(acc_a) + list(acc_b), b1.reshape(1, 64), w2p)
    acc2 = _sc_stage2(edge_index, t2, zeros8)
    return _tc_stage5(acc2, b2.reshape(1, 2))
